# baseline (speedup 1.0000x reference)
import functools

import jax
import jax.numpy as jnp
from jax import lax
from jax.experimental import pallas as pl
from jax.experimental.pallas import tpu as pltpu
from jax.experimental.pallas import tpu_sc as plsc

N = 10000
E = 320000
D = 128
H = 128
HG = 64
P = 64
OC = 3
OS = 12
OUT = OC * OS

NC = 2
NS = 16
NW = NC * NS

K = 128
KH = 64
CH = 10240
NB = CH // K
NPAD = 10112
RPT = NPAD // NS
P2 = 128
RPT2 = P2 // NS
CNR = CH * NW // 128 // 128 * 0 + 80


def _sc_agg1_body(x_hbm, srcp, dstp, ddiv, dmod, ident, zagg, zcnt2d,
                  agg_out, cnt_out,
                  sidx0, didx0, dvi, dmi, rows0, oh,
                  sidx1, didx1, dvi1, dmi1, rows1, oh1,
                  sh_agg, sh_cnt, sem0, sem1, sem2, sem3):
    c = lax.axis_index("c")
    s = lax.axis_index("s")
    tile = c * NS + s
    row0 = pl.multiple_of(s * RPT, 8)
    pltpu.sync_copy(zagg.at[pl.ds(row0, RPT)], sh_agg.at[pl.ds(row0, RPT)])
    rowc = pl.multiple_of(s * 8, 8)

    @pl.when(s < CNR // 8)
    def _():
        pltpu.sync_copy(zcnt2d.at[pl.ds(rowc, 8)], sh_cnt.at[pl.ds(rowc, 8)])

    plsc.subcore_barrier()

    def step(j2, carry):
        base = pl.multiple_of(tile * CH + j2 * 2 * KH, 8)
        pltpu.sync_copy(srcp.at[pl.ds(base, KH)], sidx0)
        pltpu.sync_copy(dstp.at[pl.ds(base, KH)], didx0)
        pltpu.sync_copy(ddiv.at[pl.ds(base, KH)], dvi)
        pltpu.sync_copy(dmod.at[pl.ds(base, KH)], dmi)
        cpx = pltpu.async_copy(x_hbm.at[sidx0], rows0, sem0)
        cpo = pltpu.async_copy(ident.at[dmi], oh, sem1)
        base1 = pl.multiple_of(base + KH, 8)
        pltpu.sync_copy(srcp.at[pl.ds(base1, KH)], sidx1)
        pltpu.sync_copy(dstp.at[pl.ds(base1, KH)], didx1)
        pltpu.sync_copy(ddiv.at[pl.ds(base1, KH)], dvi1)
        pltpu.sync_copy(dmod.at[pl.ds(base1, KH)], dmi1)
        cpx1 = pltpu.async_copy(x_hbm.at[sidx1], rows1, sem2)
        cpo1 = pltpu.async_copy(ident.at[dmi1], oh1, sem3)
        cpx.wait()
        pltpu.sync_copy(rows0, sh_agg.at[didx0], add=True)
        cpo.wait()
        pltpu.sync_copy(oh, sh_cnt.at[dvi], add=True)
        cpx1.wait()
        pltpu.sync_copy(rows1, sh_agg.at[didx1], add=True)
        cpo1.wait()
        pltpu.sync_copy(oh1, sh_cnt.at[dvi1], add=True)
        return carry

    lax.fori_loop(0, CH // (2 * KH), step, 0)
    plsc.subcore_barrier()
    pltpu.sync_copy(sh_agg.at[pl.ds(row0, RPT)],
                    agg_out.at[c, pl.ds(row0, RPT)])

    @pl.when(s < CNR // 8)
    def _():
        pltpu.sync_copy(sh_cnt.at[pl.ds(rowc, 8)],
                        cnt_out.at[c, pl.ds(rowc, 8)])


def _sc_agg2_body(z_hbm, srcp, dstcp, zagg2,
                  agg_out,
                  sidx0, sidx1, didx0, didx1, rows0, rows1,
                  sh_agg, sem0, sem1):
    c = lax.axis_index("c")
    s = lax.axis_index("s")
    tile = c * NS + s
    row0 = pl.multiple_of(s * RPT2, 8)
    pltpu.sync_copy(zagg2.at[pl.ds(row0, RPT2)],
                    sh_agg.at[pl.ds(row0, RPT2)])
    plsc.subcore_barrier()

    def step(j2, carry):
        base = pl.multiple_of(tile * CH + j2 * 2 * K, 8)
        pltpu.sync_copy(srcp.at[pl.ds(base, K)], sidx0)
        pltpu.sync_copy(dstcp.at[pl.ds(base, K)], didx0)
        cp0 = pltpu.async_copy(z_hbm.at[sidx0], rows0, sem0)
        base1 = pl.multiple_of(base + K, 8)
        pltpu.sync_copy(srcp.at[pl.ds(base1, K)], sidx1)
        pltpu.sync_copy(dstcp.at[pl.ds(base1, K)], didx1)
        cp1 = pltpu.async_copy(z_hbm.at[sidx1], rows1, sem1)
        cp0.wait()
        pltpu.sync_copy(rows0, sh_agg.at[didx0], add=True)
        cp1.wait()
        pltpu.sync_copy(rows1, sh_agg.at[didx1], add=True)
        return carry

    lax.fori_loop(0, NB // 2, step, 0)
    plsc.subcore_barrier()
    pltpu.sync_copy(sh_agg.at[pl.ds(row0, RPT2)],
                    agg_out.at[c, pl.ds(row0, RPT2)])


@functools.cache
def _make_sc_kernels():
  mesh = plsc.VectorSubcoreMesh(core_axis_name="c", subcore_axis_name="s")
  sc_agg1 = pl.kernel(
    _sc_agg1_body,
    out_type=[
        jax.ShapeDtypeStruct((NC, NPAD, D), jnp.float32),
        jax.ShapeDtypeStruct((NC, CNR, 128), jnp.float32),
    ],
    mesh=mesh,
    scratch_types=[
        pltpu.VMEM((KH,), jnp.int32),
        pltpu.VMEM((KH,), jnp.int32),
        pltpu.VMEM((KH,), jnp.int32),
        pltpu.VMEM((KH,), jnp.int32),
        pltpu.VMEM((KH, D), jnp.float32),
        pltpu.VMEM((KH, 128), jnp.float32),
        pltpu.VMEM((KH,), jnp.int32),
        pltpu.VMEM((KH,), jnp.int32),
        pltpu.VMEM((KH,), jnp.int32),
        pltpu.VMEM((KH,), jnp.int32),
        pltpu.VMEM((KH, D), jnp.float32),
        pltpu.VMEM((KH, 128), jnp.float32),
        pltpu.VMEM_SHARED((NPAD, D), jnp.float32),
        pltpu.VMEM_SHARED((CNR, 128), jnp.float32),
        pltpu.SemaphoreType.DMA,
        pltpu.SemaphoreType.DMA,
        pltpu.SemaphoreType.DMA,
        pltpu.SemaphoreType.DMA,
    ],
  )

  sc_agg2 = pl.kernel(
    _sc_agg2_body,
    out_type=[jax.ShapeDtypeStruct((NC, P2, H), jnp.float32)],
    mesh=mesh,
    scratch_types=[
        pltpu.VMEM((K,), jnp.int32),
        pltpu.VMEM((K,), jnp.int32),
        pltpu.VMEM((K,), jnp.int32),
        pltpu.VMEM((K,), jnp.int32),
        pltpu.VMEM((K, H), jnp.float32),
        pltpu.VMEM((K, H), jnp.float32),
        pltpu.VMEM_SHARED((P2, H), jnp.float32),
        pltpu.SemaphoreType.DMA,
        pltpu.SemaphoreType.DMA,
    ],
  )
  return sc_agg1, sc_agg2


def _z1_body(agg_ref, cnt_ref, x_ref, w1l_ref, w1r_ref, b1l_ref, out_ref):
    agg = agg_ref[0] + agg_ref[1]
    cnt = jnp.sum(cnt_ref[...], axis=1)
    recip = 1.0 / jnp.maximum(cnt, 1.0)
    mean = agg * recip[:, None]
    z = (jnp.dot(mean, w1l_ref[...], preferred_element_type=jnp.float32)
         + jnp.dot(x_ref[...], w1r_ref[...], preferred_element_type=jnp.float32)
         + b1l_ref[...])
    out_ref[...] = jnp.maximum(z, 0.0)


_Z1_BLK = 400


def _z1_call(agg, cnt, x, W1l, W1r, b1l):
    grid = N // _Z1_BLK
    return pl.pallas_call(
        _z1_body,
        grid=(grid,),
        in_specs=[
            pl.BlockSpec((NC, _Z1_BLK, D), lambda i: (0, i, 0)),
            pl.BlockSpec((_Z1_BLK, NC), lambda i: (i, 0)),
            pl.BlockSpec((_Z1_BLK, D), lambda i: (i, 0)),
            pl.BlockSpec((D, H), lambda i: (0, 0)),
            pl.BlockSpec((D, H), lambda i: (0, 0)),
            pl.BlockSpec((1, H), lambda i: (0, 0)),
        ],
        out_specs=pl.BlockSpec((_Z1_BLK, H), lambda i: (i, 0)),
        out_shape=jax.ShapeDtypeStruct((N, H), jnp.float32),
    )(agg, cnt, x, W1l, W1r, b1l)


def _dec_body(agg2_ref, cnt_ref, z1p_ref, w2l_ref, w2r_ref, b2l_ref,
              wih_ref, bih_ref, bhh_ref,
              w0_ref, b0_ref, w1_ref, b1_ref, w2_ref, b2_ref, w3_ref, b3_ref,
              out_ref):
    aggsum = agg2_ref[:, 0, :] + agg2_ref[:, 1, :]
    cntp = jnp.sum(cnt_ref[0, 0, :])
    recip = 1.0 / jnp.maximum(cntp, 1.0)
    mean2 = aggsum * recip
    z1p = z1p_ref[0]
    z2 = (jnp.dot(mean2, w2l_ref[...], preferred_element_type=jnp.float32)
          + jnp.dot(z1p, w2r_ref[...], preferred_element_type=jnp.float32)
          + b2l_ref[0])
    gi = lax.dot_general(z2, wih_ref[0], (((1,), (1,)), ((), ())),
                         preferred_element_type=jnp.float32) + bih_ref[0]
    bhh = bhh_ref[0]
    r = jax.nn.sigmoid(gi[:, 0:HG] + bhh[:, 0:HG])
    zg = jax.nn.sigmoid(gi[:, HG:2 * HG] + bhh[:, HG:2 * HG])
    n = jnp.tanh(gi[:, 2 * HG:] + r * bhh[:, 2 * HG:])
    h = (1.0 - zg) * n
    o = jnp.maximum(jnp.dot(h, w0_ref[0], preferred_element_type=jnp.float32)
                    + b0_ref[0], 0.0)
    o = jnp.maximum(jnp.dot(o, w1_ref[0], preferred_element_type=jnp.float32)
                    + b1_ref[0], 0.0)
    o = jnp.maximum(jnp.dot(o, w2_ref[0], preferred_element_type=jnp.float32)
                    + b2_ref[0], 0.0)
    o = jnp.dot(o, w3_ref[0], preferred_element_type=jnp.float32) + b3_ref[0]
    out_ref[0] = o


def _dec_call(agg2t, cnt64t, z1p, W2l, W2r, b2l,
              gru_Wih, gru_bih, gru_bhh,
              mW0, mb0, mW1, mb1, mW2, mb2, mW3, mb3):
    def full3(a, b):
        return pl.BlockSpec((1, a, b), lambda p: (p, 0, 0))

    def bcast(shape):
        nd = len(shape)
        return pl.BlockSpec(shape, lambda p: (0,) * nd)

    return pl.pallas_call(
        _dec_body,
        grid=(P,),
        in_specs=[
            full3(NC, H),
            full3(1, NC),
            full3(1, H),
            bcast((D, H)),
            bcast((D, H)),
            bcast((1, H)),
            full3(3 * HG, H),
            full3(1, 3 * HG),
            full3(1, 3 * HG),
            full3(HG, 128),
            full3(1, 128),
            full3(128, 64),
            full3(1, 64),
            full3(64, 32),
            full3(1, 32),
            full3(32, OUT),
            full3(1, OUT),
        ],
        out_specs=pl.BlockSpec((1, 1, OUT), lambda p: (p, 0, 0)),
        out_shape=jax.ShapeDtypeStruct((P, 1, OUT), jnp.float32),
    )(agg2t, cnt64t, z1p, W2l, W2r, b2l.reshape(1, H),
      gru_Wih, gru_bih.reshape(P, 1, 3 * HG), gru_bhh.reshape(P, 1, 3 * HG),
      mW0, mb0.reshape(P, 1, 128), mW1, mb1.reshape(P, 1, 64),
      mW2, mb2.reshape(P, 1, 32), mW3, mb3.reshape(P, 1, OUT))


def kernel(x, edge_index, W1l, b1l, W1r, W2l, b2l, W2r,
           gru_Wih, gru_Whh, gru_bih, gru_bhh,
           mW0, mb0, mW1, mb1, mW2, mb2, mW3, mb3):
    src = edge_index[0].astype(jnp.int32)
    dst = edge_index[1].astype(jnp.int32)
    perw = E // NW
    pad = CH - perw
    src2 = src.reshape(NW, perw)
    dst2 = dst.reshape(NW, perw)
    srcp = jnp.concatenate(
        [src2, jnp.zeros((NW, pad), jnp.int32)], axis=1).reshape(NW * CH)
    dstp = jnp.concatenate(
        [dst2, jnp.full((NW, pad), N, jnp.int32)], axis=1).reshape(NW * CH)
    dstc2 = jnp.minimum(dst2, P)
    dstcp = jnp.concatenate(
        [dstc2, jnp.full((NW, pad), P, jnp.int32)], axis=1).reshape(NW * CH)

    ddiv = dstp // 128
    dmod = dstp % 128
    ident = jnp.eye(128, dtype=jnp.float32)
    zagg = jnp.zeros((NPAD, D), jnp.float32)
    zcnt2d = jnp.zeros((CNR, 128), jnp.float32)
    zagg2 = jnp.zeros((P2, H), jnp.float32)

    sc_agg1, sc_agg2 = _make_sc_kernels()
    agg, cnt3 = sc_agg1(x, srcp, dstp, ddiv, dmod, ident, zagg, zcnt2d)
    cntT = cnt3.reshape(NC, CNR * 128)[:, :N].T
    z1 = _z1_call(agg[:, :N], cntT, x, W1l, W1r, b1l.reshape(1, H))
    (agg2,) = sc_agg2(z1, srcp, dstcp, zagg2)

    agg2t = agg2[:, :P].transpose(1, 0, 2)
    cnt64t = cntT[:P].reshape(P, 1, NC)
    z1p = z1[:P].reshape(P, 1, H)
    out = _dec_call(agg2t, cnt64t, z1p, W2l, W2r, b2l,
                    gru_Wih, gru_bih, gru_bhh,
                    mW0, mb0, mW1, mb1, mW2, mb2, mW3, mb3)
    return out.reshape(P, OC, OS)


# speedup vs baseline: 1.0434x; 1.0434x over previous
import functools

import jax
import jax.numpy as jnp
from jax import lax
from jax.experimental import pallas as pl
from jax.experimental.pallas import tpu as pltpu
from jax.experimental.pallas import tpu_sc as plsc

N = 10000
E = 320000
D = 128
H = 128
HG = 64
P = 64
OC = 3
OS = 12
OUT = OC * OS

NC = 2
NS = 16
NW = NC * NS

K = 128
CH = 10240
NB = CH // K
NPAD = 10112
RPT = NPAD // NS
P2 = 128
RPT2 = P2 // NS
CNR = CH * NW // 128 // 128 * 0 + 80


def _sc_agg1_body(x_hbm, srcp, dstp, ddiv, dmod, ident, zagg, zcnt2d,
                  agg_out, cnt_out,
                  sidx0, didx0, dvi, dmi, rows0, oh,
                  sh_agg, sh_cnt, sh_id, sem0, sem1):
    c = lax.axis_index("c")
    s = lax.axis_index("s")
    tile = c * NS + s
    row0 = pl.multiple_of(s * RPT, 8)
    pltpu.sync_copy(zagg.at[pl.ds(row0, RPT)], sh_agg.at[pl.ds(row0, RPT)])
    rowc = pl.multiple_of(s * 8, 8)

    @pl.when(s < CNR // 8)
    def _():
        pltpu.sync_copy(zcnt2d.at[pl.ds(rowc, 8)], sh_cnt.at[pl.ds(rowc, 8)])

    @pl.when(s == 15)
    def _():
        pltpu.sync_copy(ident, sh_id)

    plsc.subcore_barrier()

    def step(j0, carry):
        base = pl.multiple_of(tile * CH + j0 * K, 8)
        pltpu.sync_copy(srcp.at[pl.ds(base, K)], sidx0)
        pltpu.sync_copy(dstp.at[pl.ds(base, K)], didx0)
        pltpu.sync_copy(ddiv.at[pl.ds(base, K)], dvi)
        pltpu.sync_copy(dmod.at[pl.ds(base, K)], dmi)
        cpx = pltpu.async_copy(x_hbm.at[sidx0], rows0, sem0)
        cpo = pltpu.async_copy(sh_id.at[dmi], oh, sem1)
        cpx.wait()
        pltpu.sync_copy(rows0, sh_agg.at[didx0], add=True)
        cpo.wait()
        pltpu.sync_copy(oh, sh_cnt.at[dvi], add=True)
        return carry

    lax.fori_loop(0, NB, step, 0)
    plsc.subcore_barrier()
    pltpu.sync_copy(sh_agg.at[pl.ds(row0, RPT)],
                    agg_out.at[c, pl.ds(row0, RPT)])

    @pl.when(s < CNR // 8)
    def _():
        pltpu.sync_copy(sh_cnt.at[pl.ds(rowc, 8)],
                        cnt_out.at[c, pl.ds(rowc, 8)])


def _sc_agg2_body(z_hbm, srcp, dstcp, zagg2,
                  agg_out,
                  sidx0, sidx1, didx0, didx1, rows0, rows1,
                  sh_agg, sem0, sem1):
    c = lax.axis_index("c")
    s = lax.axis_index("s")
    tile = c * NS + s
    row0 = pl.multiple_of(s * RPT2, 8)
    pltpu.sync_copy(zagg2.at[pl.ds(row0, RPT2)],
                    sh_agg.at[pl.ds(row0, RPT2)])
    plsc.subcore_barrier()

    def step(j2, carry):
        base = pl.multiple_of(tile * CH + j2 * 2 * K, 8)
        pltpu.sync_copy(srcp.at[pl.ds(base, K)], sidx0)
        pltpu.sync_copy(dstcp.at[pl.ds(base, K)], didx0)
        cp0 = pltpu.async_copy(z_hbm.at[sidx0], rows0, sem0)
        base1 = pl.multiple_of(base + K, 8)
        pltpu.sync_copy(srcp.at[pl.ds(base1, K)], sidx1)
        pltpu.sync_copy(dstcp.at[pl.ds(base1, K)], didx1)
        cp1 = pltpu.async_copy(z_hbm.at[sidx1], rows1, sem1)
        cp0.wait()
        pltpu.sync_copy(rows0, sh_agg.at[didx0], add=True)
        cp1.wait()
        pltpu.sync_copy(rows1, sh_agg.at[didx1], add=True)
        return carry

    lax.fori_loop(0, NB // 2, step, 0)
    plsc.subcore_barrier()
    pltpu.sync_copy(sh_agg.at[pl.ds(row0, RPT2)],
                    agg_out.at[c, pl.ds(row0, RPT2)])


@functools.cache
def _make_sc_kernels():
  mesh = plsc.VectorSubcoreMesh(core_axis_name="c", subcore_axis_name="s")
  sc_agg1 = pl.kernel(
    _sc_agg1_body,
    out_type=[
        jax.ShapeDtypeStruct((NC, NPAD, D), jnp.float32),
        jax.ShapeDtypeStruct((NC, CNR, 128), jnp.float32),
    ],
    mesh=mesh,
    scratch_types=[
        pltpu.VMEM((K,), jnp.int32),
        pltpu.VMEM((K,), jnp.int32),
        pltpu.VMEM((K,), jnp.int32),
        pltpu.VMEM((K,), jnp.int32),
        pltpu.VMEM((K, D), jnp.float32),
        pltpu.VMEM((K, 128), jnp.float32),
        pltpu.VMEM_SHARED((NPAD, D), jnp.float32),
        pltpu.VMEM_SHARED((CNR, 128), jnp.float32),
        pltpu.VMEM_SHARED((128, 128), jnp.float32),
        pltpu.SemaphoreType.DMA,
        pltpu.SemaphoreType.DMA,
    ],
  )

  sc_agg2 = pl.kernel(
    _sc_agg2_body,
    out_type=[jax.ShapeDtypeStruct((NC, P2, H), jnp.float32)],
    mesh=mesh,
    scratch_types=[
        pltpu.VMEM((K,), jnp.int32),
        pltpu.VMEM((K,), jnp.int32),
        pltpu.VMEM((K,), jnp.int32),
        pltpu.VMEM((K,), jnp.int32),
        pltpu.VMEM((K, H), jnp.float32),
        pltpu.VMEM((K, H), jnp.float32),
        pltpu.VMEM_SHARED((P2, H), jnp.float32),
        pltpu.SemaphoreType.DMA,
        pltpu.SemaphoreType.DMA,
    ],
  )
  return sc_agg1, sc_agg2


def _z1_body(agg_ref, cnt_ref, x_ref, w1l_ref, w1r_ref, b1l_ref, out_ref):
    agg = agg_ref[0] + agg_ref[1]
    cnt = jnp.sum(cnt_ref[...], axis=1)
    recip = 1.0 / jnp.maximum(cnt, 1.0)
    mean = agg * recip[:, None]
    z = (jnp.dot(mean, w1l_ref[...], preferred_element_type=jnp.float32)
         + jnp.dot(x_ref[...], w1r_ref[...], preferred_element_type=jnp.float32)
         + b1l_ref[...])
    out_ref[...] = jnp.maximum(z, 0.0)


_Z1_BLK = 400


def _z1_call(agg, cnt, x, W1l, W1r, b1l):
    grid = N // _Z1_BLK
    return pl.pallas_call(
        _z1_body,
        grid=(grid,),
        in_specs=[
            pl.BlockSpec((NC, _Z1_BLK, D), lambda i: (0, i, 0)),
            pl.BlockSpec((_Z1_BLK, NC), lambda i: (i, 0)),
            pl.BlockSpec((_Z1_BLK, D), lambda i: (i, 0)),
            pl.BlockSpec((D, H), lambda i: (0, 0)),
            pl.BlockSpec((D, H), lambda i: (0, 0)),
            pl.BlockSpec((1, H), lambda i: (0, 0)),
        ],
        out_specs=pl.BlockSpec((_Z1_BLK, H), lambda i: (i, 0)),
        out_shape=jax.ShapeDtypeStruct((N, H), jnp.float32),
    )(agg, cnt, x, W1l, W1r, b1l)


def _dec_body(agg2_ref, cnt_ref, z1p_ref, w2l_ref, w2r_ref, b2l_ref,
              wih_ref, bih_ref, bhh_ref,
              w0_ref, b0_ref, w1_ref, b1_ref, w2_ref, b2_ref, w3_ref, b3_ref,
              out_ref):
    aggsum = agg2_ref[:, 0, :] + agg2_ref[:, 1, :]
    cntp = jnp.sum(cnt_ref[0, 0, :])
    recip = 1.0 / jnp.maximum(cntp, 1.0)
    mean2 = aggsum * recip
    z1p = z1p_ref[0]
    z2 = (jnp.dot(mean2, w2l_ref[...], preferred_element_type=jnp.float32)
          + jnp.dot(z1p, w2r_ref[...], preferred_element_type=jnp.float32)
          + b2l_ref[0])
    gi = lax.dot_general(z2, wih_ref[0], (((1,), (1,)), ((), ())),
                         preferred_element_type=jnp.float32) + bih_ref[0]
    bhh = bhh_ref[0]
    r = jax.nn.sigmoid(gi[:, 0:HG] + bhh[:, 0:HG])
    zg = jax.nn.sigmoid(gi[:, HG:2 * HG] + bhh[:, HG:2 * HG])
    n = jnp.tanh(gi[:, 2 * HG:] + r * bhh[:, 2 * HG:])
    h = (1.0 - zg) * n
    o = jnp.maximum(jnp.dot(h, w0_ref[0], preferred_element_type=jnp.float32)
                    + b0_ref[0], 0.0)
    o = jnp.maximum(jnp.dot(o, w1_ref[0], preferred_element_type=jnp.float32)
                    + b1_ref[0], 0.0)
    o = jnp.maximum(jnp.dot(o, w2_ref[0], preferred_element_type=jnp.float32)
                    + b2_ref[0], 0.0)
    o = jnp.dot(o, w3_ref[0], preferred_element_type=jnp.float32) + b3_ref[0]
    out_ref[0] = o


def _dec_call(agg2t, cnt64t, z1p, W2l, W2r, b2l,
              gru_Wih, gru_bih, gru_bhh,
              mW0, mb0, mW1, mb1, mW2, mb2, mW3, mb3):
    def full3(a, b):
        return pl.BlockSpec((1, a, b), lambda p: (p, 0, 0))

    def bcast(shape):
        nd = len(shape)
        return pl.BlockSpec(shape, lambda p: (0,) * nd)

    return pl.pallas_call(
        _dec_body,
        grid=(P,),
        in_specs=[
            full3(NC, H),
            full3(1, NC),
            full3(1, H),
            bcast((D, H)),
            bcast((D, H)),
            bcast((1, H)),
            full3(3 * HG, H),
            full3(1, 3 * HG),
            full3(1, 3 * HG),
            full3(HG, 128),
            full3(1, 128),
            full3(128, 64),
            full3(1, 64),
            full3(64, 32),
            full3(1, 32),
            full3(32, OUT),
            full3(1, OUT),
        ],
        out_specs=pl.BlockSpec((1, 1, OUT), lambda p: (p, 0, 0)),
        out_shape=jax.ShapeDtypeStruct((P, 1, OUT), jnp.float32),
    )(agg2t, cnt64t, z1p, W2l, W2r, b2l.reshape(1, H),
      gru_Wih, gru_bih.reshape(P, 1, 3 * HG), gru_bhh.reshape(P, 1, 3 * HG),
      mW0, mb0.reshape(P, 1, 128), mW1, mb1.reshape(P, 1, 64),
      mW2, mb2.reshape(P, 1, 32), mW3, mb3.reshape(P, 1, OUT))


def kernel(x, edge_index, W1l, b1l, W1r, W2l, b2l, W2r,
           gru_Wih, gru_Whh, gru_bih, gru_bhh,
           mW0, mb0, mW1, mb1, mW2, mb2, mW3, mb3):
    src = edge_index[0].astype(jnp.int32)
    dst = edge_index[1].astype(jnp.int32)
    perw = E // NW
    pad = CH - perw
    src2 = src.reshape(NW, perw)
    dst2 = dst.reshape(NW, perw)
    srcp = jnp.concatenate(
        [src2, jnp.zeros((NW, pad), jnp.int32)], axis=1).reshape(NW * CH)
    dstp = jnp.concatenate(
        [dst2, jnp.full((NW, pad), N, jnp.int32)], axis=1).reshape(NW * CH)
    dstc2 = jnp.minimum(dst2, P)
    dstcp = jnp.concatenate(
        [dstc2, jnp.full((NW, pad), P, jnp.int32)], axis=1).reshape(NW * CH)

    ddiv = dstp // 128
    dmod = dstp % 128
    ident = jnp.eye(128, dtype=jnp.float32)
    zagg = jnp.zeros((NPAD, D), jnp.float32)
    zcnt2d = jnp.zeros((CNR, 128), jnp.float32)
    zagg2 = jnp.zeros((P2, H), jnp.float32)

    sc_agg1, sc_agg2 = _make_sc_kernels()
    agg, cnt3 = sc_agg1(x, srcp, dstp, ddiv, dmod, ident, zagg, zcnt2d)
    cntT = cnt3.reshape(NC, CNR * 128)[:, :N].T
    z1 = _z1_call(agg[:, :N], cntT, x, W1l, W1r, b1l.reshape(1, H))
    (agg2,) = sc_agg2(z1, srcp, dstcp, zagg2)

    agg2t = agg2[:, :P].transpose(1, 0, 2)
    cnt64t = cntT[:P].reshape(P, 1, NC)
    z1p = z1[:P].reshape(P, 1, H)
    out = _dec_call(agg2t, cnt64t, z1p, W2l, W2r, b2l,
                    gru_Wih, gru_bih, gru_bhh,
                    mW0, mb0, mW1, mb1, mW2, mb2, mW3, mb3)
    return out.reshape(P, OC, OS)
